# baseline (speedup 1.0000x reference)
import jax
import jax.numpy as jnp
from jax import lax
from jax.experimental import pallas as pl
from jax.experimental.pallas import tpu as pltpu
from jax.experimental.pallas import tpu_sc as plsc

_B = 4096
_D = 64
_NC, _NS = 2, 16
_NW = _NC * _NS
_TPW = _B // _NW
_CH = 32
_NCHUNK = _TPW // _CH
_RPT = 8
_MARGIN = 1.0
_L = 16
_W = 128

_NENT = 1000000
_NBLK = _NENT // 128
_TAIL0 = _NBLK * 128
_BPW = 248
_NBUF = 4

_BLS = 8192


def _stats_body(e_ref, p_ref, s2p_ref, dot_ref):
    e = e_ref[...]
    p = p_ref[...]
    s2p_ref[...] = jnp.sum(p * p, axis=0)
    dot_ref[...] = jnp.sum(e * p, axis=0)


def _p_stats(e, p):
    eT = jnp.swapaxes(e, 0, 1)
    pT = jnp.swapaxes(p, 0, 1)
    n = eT.shape[1]
    return pl.pallas_call(
        _stats_body,
        out_shape=[jax.ShapeDtypeStruct((n,), jnp.float32),
                   jax.ShapeDtypeStruct((n,), jnp.float32)],
        grid=(pl.cdiv(n, _BLS),),
        in_specs=[pl.BlockSpec((_D, _BLS), lambda j: (0, j)),
                  pl.BlockSpec((_D, _BLS), lambda j: (0, j))],
        out_specs=[pl.BlockSpec((_BLS,), lambda j: (j,)),
                   pl.BlockSpec((_BLS,), lambda j: (j,))],
    )(eT, pT)


def _rel_body(eT_hbm, s2p_hbm, dot_hbm, tail_hbm, out_hbm,
              t0, t1, t2, t3, r0, r1, r2, r3,
              sb0, sb1, sb2, sb3, db0, db1, db2, db3,
              is0, is1, is2, is3, os0, os1, os2, os3):
    wid = lax.axis_index("s") * _NC + lax.axis_index("c")
    tiles = (t0, t1, t2, t3)
    rows = (r0, r1, r2, r3)
    sbs = (sb0, sb1, sb2, sb3)
    dbs = (db0, db1, db2, db3)
    isems = (is0, is1, is2, is3)
    osems = (os0, os1, os2, os3)
    lanes = lax.iota(jnp.int32, _L)

    def blk(off):
        return jnp.minimum(wid * _BPW + off, _NBLK - 1)

    def issue_in(off, buf):
        col = pl.multiple_of(blk(off) * 128, 128)
        pltpu.async_copy(eT_hbm.at[:, pl.ds(col, 128)], tiles[buf],
                         isems[buf])
        pltpu.async_copy(s2p_hbm.at[pl.ds(col, 128)], sbs[buf], isems[buf])
        pltpu.async_copy(dot_hbm.at[pl.ds(col, 128)], dbs[buf], isems[buf])

    def wait_in(buf):
        pltpu.make_async_copy(eT_hbm.at[:, pl.ds(0, 128)], tiles[buf],
                              isems[buf]).wait()
        pltpu.make_async_copy(s2p_hbm.at[pl.ds(0, 128)], sbs[buf],
                              isems[buf]).wait()
        pltpu.make_async_copy(dot_hbm.at[pl.ds(0, 128)], dbs[buf],
                              isems[buf]).wait()

    def issue_out(off, buf):
        ro = pl.multiple_of(blk(off) * 128, 128)
        pltpu.async_copy(rows[buf], out_hbm.at[pl.ds(ro, 128), :],
                         osems[buf])

    def wait_out(buf):
        pltpu.make_async_copy(rows[buf], out_hbm.at[pl.ds(0, 128), :],
                              osems[buf]).wait()

    def extract(buf, ncol16):
        def grp(c16, carry):
            s16 = sbs[buf][pl.ds(c16 * _L, _L)]
            d16 = dbs[buf][pl.ds(c16 * _L, _L)]
            for cs in range(_L):
                c = c16 * _L + cs
                idx_col = jnp.full((_L,), 0, jnp.int32) + c
                for g in range(_D // _L):
                    v = plsc.load_gather(tiles[buf],
                                         [lanes + g * _L, idx_col])
                    rows[buf][c, pl.ds(g * _L, _L)] = v
                sv = jnp.where(
                    lanes == 0, jnp.full((_L,), s16[cs], jnp.float32),
                    jnp.where(lanes == 1,
                              jnp.full((_L,), d16[cs], jnp.float32), 0.0))
                rows[buf][c, pl.ds(_D, _L)] = sv
                rows[buf][c, pl.ds(_D + _L, _L)] = jnp.zeros((_L,),
                                                             jnp.float32)
                rows[buf][c, pl.ds(_D + 2 * _L, _L)] = jnp.zeros(
                    (_L,), jnp.float32)
                rows[buf][c, pl.ds(_D + 3 * _L, _L)] = jnp.zeros(
                    (_L,), jnp.float32)
            return carry

        lax.fori_loop(0, ncol16, grp, 0)

    for off0 in range(_NBUF):
        issue_in(off0, off0)

    def step(j, carry):
        for buf in range(_NBUF):
            off = j * _NBUF + buf
            wait_in(buf)

            @pl.when(j > 0)
            def _():
                wait_out(buf)

            extract(buf, 8)
            issue_out(off, buf)

            @pl.when(off + _NBUF < _BPW)
            def _():
                issue_in(off + _NBUF, buf)

        return carry

    lax.fori_loop(0, _BPW // _NBUF, step, 0)
    for buf in range(_NBUF):
        wait_out(buf)

    @pl.when(wid == _NW - 1)
    def _():
        pltpu.sync_copy(tail_hbm,
                        out_hbm.at[pl.ds(_TAIL0, _NENT - _TAIL0), :])


@jax.jit
def _sc_relayout(eT, s2p, dotep, tail):
    mesh = plsc.VectorSubcoreMesh(core_axis_name="c", subcore_axis_name="s",
                                  num_cores=_NC, num_subcores=_NS)
    f = pl.kernel(
        _rel_body,
        out_type=jax.ShapeDtypeStruct((_NENT, _W), jnp.float32),
        mesh=mesh,
        compiler_params=pltpu.CompilerParams(needs_layout_passes=False),
        scratch_types=(
            [pltpu.VMEM((_D, 128), jnp.float32)] * _NBUF
            + [pltpu.VMEM((128, _W), jnp.float32)] * _NBUF
            + [pltpu.VMEM((128,), jnp.float32)] * (2 * _NBUF)
            + [pltpu.SemaphoreType.DMA] * (2 * _NBUF)
        ),
    )
    return f(eT, s2p, dotep, tail)


def _rsqrt(x):
    i = lax.bitcast_convert_type(x, jnp.int32)
    i = jnp.int32(0x5F3759DF) - lax.shift_right_logical(i, 1)
    y = lax.bitcast_convert_type(i, jnp.float32)
    xh = x * 0.5
    for _ in range(3):
        y = y * (1.5 - xh * y * y)
    return y


def _sumall(v):
    return jnp.full((_L,), jnp.sum(v), jnp.float32)


def _scale(s2):
    norm = s2 * _rsqrt(s2)
    return jnp.minimum(1.0, 1.0 / (norm + 1e-7))


def _sc_body(entidx_hbm, relidx_hbm, eE_hbm, rE_hbm, rP_hbm,
             out_hbm, ei00, ei01, ei10, ei11, ri0, ri1,
             eE_v, rE_v, rP_v, out_v, sem0, sem1):
    wid = lax.axis_index("s") * _NC + lax.axis_index("c")
    sems = (sem0, sem1)
    eidx = ((ei00, ei01), (ei10, ei11))
    ridx = (ri0, ri1)
    handles = {}

    def issue(chunk, buf):
        base_t = wid * _TPW + chunk * _CH
        hs = []
        for j in range(2):
            pltpu.sync_copy(
                entidx_hbm.at[pl.ds(base_t * _RPT + j * 128, 128)],
                eidx[buf][j])
            dst_e = eE_v.at[buf, pl.ds(j * 128, 128), :]
            hs.append(pltpu.async_copy(eE_hbm.at[eidx[buf][j]], dst_e,
                                       sems[buf]))
        pltpu.sync_copy(relidx_hbm.at[pl.ds(base_t, _CH)], ridx[buf])
        hs.append(pltpu.async_copy(rE_hbm.at[ridx[buf]], rE_v.at[buf],
                                   sems[buf]))
        hs.append(pltpu.async_copy(rP_hbm.at[ridx[buf]], rP_v.at[buf],
                                   sems[buf]))
        handles[buf] = hs

    def row(ref, r0):
        return [ref[r0, pl.ds(c * _L, _L)] for c in range(_D // _L)]

    acc = jnp.zeros((_L,), jnp.float32)
    issue(0, 0)
    for chunk in range(_NCHUNK):
        buf = chunk % 2
        if chunk + 1 < _NCHUNK:
            issue(chunk + 1, 1 - buf)
        for h in handles[buf]:
            h.wait()
        eEb = eE_v.at[buf]
        rEb, rPb = rE_v.at[buf], rP_v.at[buf]

        def trip(i, acc):
            aE, s_ent = [], []
            for k in range(_RPT):
                r0 = i * _RPT + k
                e = row(eEb, r0)
                s2e = e[0] * e[0]
                for c in range(1, _D // _L):
                    s2e += e[c] * e[c]
                ae = _scale(_sumall(s2e))
                aE.append(ae)
                sv = eEb[r0, pl.ds(_D, _L)]
                apk = _scale(jnp.full((_L,), sv[0], jnp.float32))
                dk = jnp.full((_L,), sv[1], jnp.float32)
                s_ent.append(ae * apk * dk)
            re = row(rEb, i)
            rp = row(rPb, i)
            s2re = re[0] * re[0]
            s2rp = rp[0] * rp[0]
            for c in range(1, _D // _L):
                s2re += re[c] * re[c]
                s2rp += rp[c] * rp[c]
            ar = _scale(_sumall(s2re))
            arp = _scale(_sumall(s2rp))
            rv = [re[c] * ar + 1e-6 for c in range(_D // _L)]
            rps = [rp[c] * arp for c in range(_D // _L)]
            dists = []
            for (hk, tk) in ((0, 1), (2, 5), (3, 6), (4, 7)):
                dS = s_ent[hk] - s_ent[tk]
                eh = row(eEb, i * _RPT + hk)
                et = row(eEb, i * _RPT + tk)
                ds2 = None
                for c in range(_D // _L):
                    dv = (eh[c] * aE[hk] - et[c] * aE[tk]
                          + rps[c] * dS + rv[c])
                    ds2 = dv * dv if ds2 is None else ds2 + dv * dv
                s2 = _sumall(ds2)
                dists.append(s2 * _rsqrt(s2))
            neg_mean = (dists[1] + dists[2] + dists[3]) * (1.0 / 3.0)
            loss = jnp.maximum(dists[0] - neg_mean + _MARGIN, 0.0)
            return acc + loss

        acc = lax.fori_loop(0, _CH, trip, acc)

    for g in range(_W // _L):
        out_v[pl.ds(g * _L, _L)] = acc if g == 0 else jnp.zeros((_L,),
                                                                jnp.float32)
    pltpu.sync_copy(out_v, out_hbm.at[wid])


@jax.jit
def _sc_call(entidx, relidx, eE, rE, rP):
    mesh = plsc.VectorSubcoreMesh(core_axis_name="c", subcore_axis_name="s",
                                  num_cores=_NC, num_subcores=_NS)
    f = pl.kernel(
        _sc_body,
        out_type=jax.ShapeDtypeStruct((_NW, _W), jnp.float32),
        mesh=mesh,
        compiler_params=pltpu.CompilerParams(needs_layout_passes=False),
        scratch_types=[
            pltpu.VMEM((128,), jnp.int32),
            pltpu.VMEM((128,), jnp.int32),
            pltpu.VMEM((128,), jnp.int32),
            pltpu.VMEM((128,), jnp.int32),
            pltpu.VMEM((_CH,), jnp.int32),
            pltpu.VMEM((_CH,), jnp.int32),
            pltpu.VMEM((2, _CH * _RPT, _W), jnp.float32),
            pltpu.VMEM((2, _CH, _W), jnp.float32),
            pltpu.VMEM((2, _CH, _W), jnp.float32),
            pltpu.VMEM((_W,), jnp.float32),
            pltpu.SemaphoreType.DMA,
            pltpu.SemaphoreType.DMA,
        ],
    )
    return f(entidx, relidx, eE, rE, rP)


def kernel(triplets, neg, entityEmb, entityEmbP, relationEmb, relationEmbP):
    h = triplets[:, 0:1].astype(jnp.int32)
    t = triplets[:, 2:3].astype(jnp.int32)
    r = triplets[:, 1].astype(jnp.int32)
    nh = neg[:, :, 0].astype(jnp.int32)
    nt = neg[:, :, 2].astype(jnp.int32)
    ent = jnp.concatenate([h, t, nh, nt], axis=1).reshape(-1)
    s2p, dotep = _p_stats(entityEmb, entityEmbP)
    ntail = _NENT - _TAIL0
    tail = jnp.concatenate(
        [entityEmb[_TAIL0:], s2p[_TAIL0:, None], dotep[_TAIL0:, None],
         jnp.zeros((ntail, _W - _D - 2), jnp.float32)], axis=1)
    eE = _sc_relayout(jnp.swapaxes(entityEmb, 0, 1), s2p, dotep, tail)
    zpad = jnp.zeros((relationEmb.shape[0], _W - _D), jnp.float32)
    rE = jnp.concatenate([relationEmb, zpad], axis=1)
    rP = jnp.concatenate([relationEmbP, zpad], axis=1)
    out = _sc_call(ent, r, eE, rE, rP)
    return jnp.sum(out[:, 0]) / _B


# speedup vs baseline: 1.0263x; 1.0263x over previous
import jax
import jax.numpy as jnp
from jax import lax
from jax.experimental import pallas as pl
from jax.experimental.pallas import tpu as pltpu
from jax.experimental.pallas import tpu_sc as plsc

_B = 4096
_D = 64
_NC, _NS = 2, 16
_NW = _NC * _NS
_TPW = _B // _NW
_CH = 32
_NCHUNK = _TPW // _CH
_RPT = 8
_MARGIN = 1.0
_L = 16
_W = 128

_NENT = 1000000
_NBLK = _NENT // 128
_TAIL0 = _NBLK * 128
_BPW = 248
_NBUF = 4

_BLS = 8192


def _stats_body(e_ref, p_ref, s2p_ref, dot_ref):
    e = e_ref[...]
    p = p_ref[...]
    s2p_ref[...] = jnp.sum(p * p, axis=0)
    dot_ref[...] = jnp.sum(e * p, axis=0)


def _p_stats(e, p):
    eT = jnp.swapaxes(e, 0, 1)
    pT = jnp.swapaxes(p, 0, 1)
    n = eT.shape[1]
    return pl.pallas_call(
        _stats_body,
        out_shape=[jax.ShapeDtypeStruct((n,), jnp.float32),
                   jax.ShapeDtypeStruct((n,), jnp.float32)],
        grid=(pl.cdiv(n, _BLS),),
        in_specs=[pl.BlockSpec((_D, _BLS), lambda j: (0, j)),
                  pl.BlockSpec((_D, _BLS), lambda j: (0, j))],
        out_specs=[pl.BlockSpec((_BLS,), lambda j: (j,)),
                   pl.BlockSpec((_BLS,), lambda j: (j,))],
    )(eT, pT)


def _rel_body(eT_hbm, s2p_hbm, dot_hbm, tail_hbm, out_hbm,
              t0, t1, t2, t3, r0, r1, r2, r3,
              sb0, sb1, sb2, sb3, db0, db1, db2, db3,
              is0, is1, is2, is3, os0, os1, os2, os3):
    wid = lax.axis_index("s") * _NC + lax.axis_index("c")
    tiles = (t0, t1, t2, t3)
    rows = (r0, r1, r2, r3)
    sbs = (sb0, sb1, sb2, sb3)
    dbs = (db0, db1, db2, db3)
    isems = (is0, is1, is2, is3)
    osems = (os0, os1, os2, os3)
    lanes = lax.iota(jnp.int32, _L)

    def blk(off):
        return jnp.minimum(wid * _BPW + off, _NBLK - 1)

    def issue_in(off, buf):
        col = pl.multiple_of(blk(off) * 128, 128)
        pltpu.async_copy(eT_hbm.at[:, pl.ds(col, 128)], tiles[buf],
                         isems[buf])
        pltpu.async_copy(s2p_hbm.at[pl.ds(col, 128)], sbs[buf], isems[buf])
        pltpu.async_copy(dot_hbm.at[pl.ds(col, 128)], dbs[buf], isems[buf])

    def wait_in(buf):
        pltpu.make_async_copy(eT_hbm.at[:, pl.ds(0, 128)], tiles[buf],
                              isems[buf]).wait()
        pltpu.make_async_copy(s2p_hbm.at[pl.ds(0, 128)], sbs[buf],
                              isems[buf]).wait()
        pltpu.make_async_copy(dot_hbm.at[pl.ds(0, 128)], dbs[buf],
                              isems[buf]).wait()

    def issue_out(off, buf):
        ro = pl.multiple_of(blk(off) * 128, 128)
        pltpu.async_copy(rows[buf], out_hbm.at[pl.ds(ro, 128), :],
                         osems[buf])

    def wait_out(buf):
        pltpu.make_async_copy(rows[buf], out_hbm.at[pl.ds(0, 128), :],
                              osems[buf]).wait()

    def extract(buf, ncol16):
        def grp(c16, carry):
            s16 = sbs[buf][pl.ds(c16 * _L, _L)]
            d16 = dbs[buf][pl.ds(c16 * _L, _L)]
            for cs in range(_L):
                c = c16 * _L + cs
                idx_col = jnp.full((_L,), 0, jnp.int32) + c
                for g in range(_D // _L):
                    v = plsc.load_gather(tiles[buf],
                                         [lanes + g * _L, idx_col])
                    rows[buf][c, pl.ds(g * _L, _L)] = v
                sv = jnp.where(
                    lanes == 0, jnp.full((_L,), s16[cs], jnp.float32),
                    jnp.where(lanes == 1,
                              jnp.full((_L,), d16[cs], jnp.float32), 0.0))
                rows[buf][c, pl.ds(_D, _L)] = sv
                rows[buf][c, pl.ds(_D + _L, _L)] = jnp.zeros((_L,),
                                                             jnp.float32)
                rows[buf][c, pl.ds(_D + 2 * _L, _L)] = jnp.zeros(
                    (_L,), jnp.float32)
                rows[buf][c, pl.ds(_D + 3 * _L, _L)] = jnp.zeros(
                    (_L,), jnp.float32)
            return carry

        lax.fori_loop(0, ncol16, grp, 0)

    for off0 in range(_NBUF):
        issue_in(off0, off0)

    def step(j, carry):
        for buf in range(_NBUF):
            off = j * _NBUF + buf
            wait_in(buf)

            @pl.when(j > 0)
            def _():
                wait_out(buf)

            extract(buf, 8)
            issue_out(off, buf)

            @pl.when(off + _NBUF < _BPW)
            def _():
                issue_in(off + _NBUF, buf)

        return carry

    lax.fori_loop(0, _BPW // _NBUF, step, 0)
    for buf in range(_NBUF):
        wait_out(buf)

    @pl.when(wid == _NW - 1)
    def _():
        pltpu.sync_copy(tail_hbm,
                        out_hbm.at[pl.ds(_TAIL0, _NENT - _TAIL0), :])


@jax.jit
def _sc_relayout(eT, s2p, dotep, tail):
    mesh = plsc.VectorSubcoreMesh(core_axis_name="c", subcore_axis_name="s",
                                  num_cores=_NC, num_subcores=_NS)
    f = pl.kernel(
        _rel_body,
        out_type=jax.ShapeDtypeStruct((_NENT, _W), jnp.float32),
        mesh=mesh,
        compiler_params=pltpu.CompilerParams(needs_layout_passes=False),
        scratch_types=(
            [pltpu.VMEM((_D, 128), jnp.float32)] * _NBUF
            + [pltpu.VMEM((128, _W), jnp.float32)] * _NBUF
            + [pltpu.VMEM((128,), jnp.float32)] * (2 * _NBUF)
            + [pltpu.SemaphoreType.DMA] * (2 * _NBUF)
        ),
    )
    return f(eT, s2p, dotep, tail)


def _rsqrt(x):
    i = lax.bitcast_convert_type(x, jnp.int32)
    i = jnp.int32(0x5F3759DF) - lax.shift_right_logical(i, 1)
    y = lax.bitcast_convert_type(i, jnp.float32)
    xh = x * 0.5
    for _ in range(3):
        y = y * (1.5 - xh * y * y)
    return y


def _sumall(v):
    return jnp.full((_L,), jnp.sum(v), jnp.float32)


def _scale(s2):
    norm = s2 * _rsqrt(s2)
    return jnp.minimum(1.0, 1.0 / (norm + 1e-7))


def _sc_body(entidx_hbm, relidx_hbm, eE_hbm, rE_hbm, rP_hbm,
             out_hbm, ei00, ei01, ei10, ei11, ri0, ri1,
             eE_v, rE_v, rP_v, out_v, sem0, sem1):
    wid = lax.axis_index("s") * _NC + lax.axis_index("c")
    sems = (sem0, sem1)
    eidx = ((ei00, ei01), (ei10, ei11))
    ridx = (ri0, ri1)
    handles = {}

    def issue(chunk, buf):
        base_t = wid * _TPW + chunk * _CH
        hs = []
        for j in range(2):
            pltpu.sync_copy(
                entidx_hbm.at[pl.ds(base_t * _RPT + j * 128, 128)],
                eidx[buf][j])
            dst_e = eE_v.at[buf, pl.ds(j * 128, 128), :]
            hs.append(pltpu.async_copy(eE_hbm.at[eidx[buf][j]], dst_e,
                                       sems[buf]))
        pltpu.sync_copy(relidx_hbm.at[pl.ds(base_t, _CH)], ridx[buf])
        hs.append(pltpu.async_copy(rE_hbm.at[ridx[buf]], rE_v.at[buf],
                                   sems[buf]))
        hs.append(pltpu.async_copy(rP_hbm.at[ridx[buf]], rP_v.at[buf],
                                   sems[buf]))
        handles[buf] = hs

    def row(ref, r0):
        return [ref[r0, pl.ds(c * _L, _L)] for c in range(_D // _L)]

    acc = jnp.zeros((_L,), jnp.float32)
    issue(0, 0)
    for chunk in range(_NCHUNK):
        buf = chunk % 2
        if chunk + 1 < _NCHUNK:
            issue(chunk + 1, 1 - buf)
        for h in handles[buf]:
            h.wait()
        eEb = eE_v.at[buf]
        rEb, rPb = rE_v.at[buf], rP_v.at[buf]

        def trip(i, acc):
            aE, s_ent = [], []
            for k in range(_RPT):
                r0 = i * _RPT + k
                e = row(eEb, r0)
                s2e = e[0] * e[0]
                for c in range(1, _D // _L):
                    s2e += e[c] * e[c]
                ae = _scale(_sumall(s2e))
                aE.append(ae)
                sv = eEb[r0, pl.ds(_D, _L)]
                apk = _scale(jnp.full((_L,), sv[0], jnp.float32))
                dk = jnp.full((_L,), sv[1], jnp.float32)
                s_ent.append(ae * apk * dk)
            re = row(rEb, i)
            rp = row(rPb, i)
            s2re = re[0] * re[0]
            s2rp = rp[0] * rp[0]
            for c in range(1, _D // _L):
                s2re += re[c] * re[c]
                s2rp += rp[c] * rp[c]
            ar = _scale(_sumall(s2re))
            arp = _scale(_sumall(s2rp))
            rv = [re[c] * ar + 1e-6 for c in range(_D // _L)]
            rps = [rp[c] * arp for c in range(_D // _L)]
            dists = []
            for (hk, tk) in ((0, 1), (2, 5), (3, 6), (4, 7)):
                dS = s_ent[hk] - s_ent[tk]
                eh = row(eEb, i * _RPT + hk)
                et = row(eEb, i * _RPT + tk)
                ds2 = None
                for c in range(_D // _L):
                    dv = (eh[c] * aE[hk] - et[c] * aE[tk]
                          + rps[c] * dS + rv[c])
                    ds2 = dv * dv if ds2 is None else ds2 + dv * dv
                s2 = _sumall(ds2)
                dists.append(s2 * _rsqrt(s2))
            neg_mean = (dists[1] + dists[2] + dists[3]) * (1.0 / 3.0)
            loss = jnp.maximum(dists[0] - neg_mean + _MARGIN, 0.0)
            return acc + loss

        acc = lax.fori_loop(0, _CH, trip, acc)

    for g in range(_W // _L):
        out_v[pl.ds(g * _L, _L)] = acc if g == 0 else jnp.zeros((_L,),
                                                                jnp.float32)
    pltpu.sync_copy(out_v, out_hbm.at[wid])


@jax.jit
def _sc_call(entidx, relidx, eE, rE, rP):
    mesh = plsc.VectorSubcoreMesh(core_axis_name="c", subcore_axis_name="s",
                                  num_cores=_NC, num_subcores=_NS)
    f = pl.kernel(
        _sc_body,
        out_type=jax.ShapeDtypeStruct((_NW, _W), jnp.float32),
        mesh=mesh,
        compiler_params=pltpu.CompilerParams(needs_layout_passes=False),
        scratch_types=[
            pltpu.VMEM((128,), jnp.int32),
            pltpu.VMEM((128,), jnp.int32),
            pltpu.VMEM((128,), jnp.int32),
            pltpu.VMEM((128,), jnp.int32),
            pltpu.VMEM((_CH,), jnp.int32),
            pltpu.VMEM((_CH,), jnp.int32),
            pltpu.VMEM((2, _CH * _RPT, _W), jnp.float32),
            pltpu.VMEM((2, _CH, _W), jnp.float32),
            pltpu.VMEM((2, _CH, _W), jnp.float32),
            pltpu.VMEM((_W,), jnp.float32),
            pltpu.SemaphoreType.DMA,
            pltpu.SemaphoreType.DMA,
        ],
    )
    return f(entidx, relidx, eE, rE, rP)


def kernel(triplets, neg, entityEmb, entityEmbP, relationEmb, relationEmbP):
    h = triplets[:, 0:1].astype(jnp.int32)
    t = triplets[:, 2:3].astype(jnp.int32)
    r = triplets[:, 1].astype(jnp.int32)
    nh = neg[:, :, 0].astype(jnp.int32)
    nt = neg[:, :, 2].astype(jnp.int32)
    ent = jnp.concatenate([h, t, nh, nt], axis=1).reshape(-1)
    s2p, dotep = _p_stats(entityEmb, entityEmbP)
    eE = jnp.concatenate(
        [entityEmb, s2p[:, None], dotep[:, None],
         jnp.zeros((_NENT, _W - _D - 2), jnp.float32)], axis=1)
    zpad = jnp.zeros((relationEmb.shape[0], _W - _D), jnp.float32)
    rE = jnp.concatenate([relationEmb, zpad], axis=1)
    rP = jnp.concatenate([relationEmbP, zpad], axis=1)
    out = _sc_call(ent, r, eE, rE, rP)
    return jnp.sum(out[:, 0]) / _B


# speedup vs baseline: 2.3336x; 2.2737x over previous
import functools

import jax
import jax.numpy as jnp
from jax import lax
from jax.experimental import pallas as pl
from jax.experimental.pallas import tpu as pltpu
from jax.experimental.pallas import tpu_sc as plsc

_B = 4096
_D = 64
_NC, _NS = 2, 16
_NW = _NC * _NS
_TPW = _B // _NW
_CH = 32
_NCHUNK = _TPW // _CH
_RPT = 8
_MARGIN = 1.0
_L = 16


_BLS = 8192


def _stats_body(e_ref, p_ref, s2p_ref, dot_ref):
    e = e_ref[...]
    p = p_ref[...]
    s2p_ref[...] = jnp.sum(p * p, axis=0)
    dot_ref[...] = jnp.sum(e * p, axis=0)


def _p_stats(e, p):
    eT = jnp.swapaxes(e, 0, 1)
    pT = jnp.swapaxes(p, 0, 1)
    n = eT.shape[1]
    return pl.pallas_call(
        _stats_body,
        out_shape=[jax.ShapeDtypeStruct((n,), jnp.float32),
                   jax.ShapeDtypeStruct((n,), jnp.float32)],
        grid=(pl.cdiv(n, _BLS),),
        in_specs=[pl.BlockSpec((_D, _BLS), lambda j: (0, j)),
                  pl.BlockSpec((_D, _BLS), lambda j: (0, j))],
        out_specs=[pl.BlockSpec((_BLS,), lambda j: (j,)),
                   pl.BlockSpec((_BLS,), lambda j: (j,))],
    )(eT, pT)


def _rsqrt(x):
    i = lax.bitcast_convert_type(x, jnp.int32)
    i = jnp.int32(0x5F3759DF) - lax.shift_right_logical(i, 1)
    y = lax.bitcast_convert_type(i, jnp.float32)
    xh = x * 0.5
    for _ in range(3):
        y = y * (1.5 - xh * y * y)
    return y


def _sumall(v):
    return jnp.full((_L,), jnp.sum(v), jnp.float32)


def _scale(s2):
    norm = s2 * _rsqrt(s2)
    return jnp.minimum(1.0, 1.0 / (norm + 1e-7))


def _sc_body(entidx_hbm, relidx_hbm, eE_hbm, s2p_hbm, dot_hbm, rE_hbm, rP_hbm,
             out_hbm, ei00, ei01, ei10, ei11, ri0, ri1,
             eE_v, s2p_v, dot_v, rE_v, rP_v, out_v, sem0, sem1):
    wid = lax.axis_index("s") * _NC + lax.axis_index("c")
    sems = (sem0, sem1)
    eidx = ((ei00, ei01), (ei10, ei11))
    ridx = (ri0, ri1)
    handles = {}

    def issue(chunk, buf):
        base_t = wid * _TPW + chunk * _CH
        hs = []
        for j in range(2):
            pltpu.sync_copy(
                entidx_hbm.at[pl.ds(base_t * _RPT + j * 128, 128)],
                eidx[buf][j])
            dst_e = eE_v.at[buf, pl.ds(j * 128, 128), :]
            dst_s = s2p_v.at[buf, pl.ds(j * 128, 128)]
            dst_d = dot_v.at[buf, pl.ds(j * 128, 128)]
            hs.append(pltpu.async_copy(eE_hbm.at[eidx[buf][j]], dst_e,
                                       sems[buf]))
            hs.append(pltpu.async_copy(s2p_hbm.at[eidx[buf][j]], dst_s,
                                       sems[buf]))
            hs.append(pltpu.async_copy(dot_hbm.at[eidx[buf][j]], dst_d,
                                       sems[buf]))
        pltpu.sync_copy(relidx_hbm.at[pl.ds(base_t, _CH)], ridx[buf])
        hs.append(pltpu.async_copy(rE_hbm.at[ridx[buf]], rE_v.at[buf],
                                   sems[buf]))
        hs.append(pltpu.async_copy(rP_hbm.at[ridx[buf]], rP_v.at[buf],
                                   sems[buf]))
        handles[buf] = hs

    def row(ref, r0):
        return [ref[r0, pl.ds(c * _L, _L)] for c in range(_D // _L)]

    acc = jnp.zeros((_L,), jnp.float32)
    issue(0, 0)
    for chunk in range(_NCHUNK):
        buf = chunk % 2
        if chunk + 1 < _NCHUNK:
            issue(chunk + 1, 1 - buf)
        for h in handles[buf]:
            h.wait()
        eEb = eE_v.at[buf]
        s2pb, dotb = s2p_v.at[buf], dot_v.at[buf]
        rEb, rPb = rE_v.at[buf], rP_v.at[buf]

        def trip(i, acc):
            s2p8 = s2pb[pl.ds(i * _RPT, _L)]
            dot8 = dotb[pl.ds(i * _RPT, _L)]
            ap8 = _scale(s2p8)
            aE, s_ent = [], []
            for k in range(_RPT):
                r0 = i * _RPT + k
                e = row(eEb, r0)
                s2e = e[0] * e[0]
                for c in range(1, _D // _L):
                    s2e += e[c] * e[c]
                ae = _scale(_sumall(s2e))
                aE.append(ae)
                apk = jnp.full((_L,), ap8[k], jnp.float32)
                dk = jnp.full((_L,), dot8[k], jnp.float32)
                s_ent.append(ae * apk * dk)
            re = row(rEb, i)
            rp = row(rPb, i)
            s2re = re[0] * re[0]
            s2rp = rp[0] * rp[0]
            for c in range(1, _D // _L):
                s2re += re[c] * re[c]
                s2rp += rp[c] * rp[c]
            ar = _scale(_sumall(s2re))
            arp = _scale(_sumall(s2rp))
            rv = [re[c] * ar + 1e-6 for c in range(_D // _L)]
            rps = [rp[c] * arp for c in range(_D // _L)]
            dists = []
            for (hk, tk) in ((0, 1), (2, 5), (3, 6), (4, 7)):
                dS = s_ent[hk] - s_ent[tk]
                eh = row(eEb, i * _RPT + hk)
                et = row(eEb, i * _RPT + tk)
                ds2 = None
                for c in range(_D // _L):
                    dv = (eh[c] * aE[hk] - et[c] * aE[tk]
                          + rps[c] * dS + rv[c])
                    ds2 = dv * dv if ds2 is None else ds2 + dv * dv
                s2 = _sumall(ds2)
                dists.append(s2 * _rsqrt(s2))
            neg_mean = (dists[1] + dists[2] + dists[3]) * (1.0 / 3.0)
            loss = jnp.maximum(dists[0] - neg_mean + _MARGIN, 0.0)
            return acc + loss

        acc = lax.fori_loop(0, _CH, trip, acc)

    out_v[...] = acc
    pltpu.sync_copy(out_v, out_hbm.at[wid])


@jax.jit
def _sc_call(entidx, relidx, eE, s2p, dotep, rE, rP):
    mesh = plsc.VectorSubcoreMesh(core_axis_name="c", subcore_axis_name="s",
                                  num_cores=_NC, num_subcores=_NS)
    f = pl.kernel(
        _sc_body,
        out_type=jax.ShapeDtypeStruct((_NW, _L), jnp.float32),
        mesh=mesh,
        compiler_params=pltpu.CompilerParams(needs_layout_passes=False,
                                             use_tc_tiling_on_sc=False),
        scratch_types=[
            pltpu.VMEM((128,), jnp.int32),
            pltpu.VMEM((128,), jnp.int32),
            pltpu.VMEM((128,), jnp.int32),
            pltpu.VMEM((128,), jnp.int32),
            pltpu.VMEM((_CH,), jnp.int32),
            pltpu.VMEM((_CH,), jnp.int32),
            pltpu.VMEM((2, _CH * _RPT, _D), jnp.float32),
            pltpu.VMEM((2, _CH * _RPT + _L, ), jnp.float32),
            pltpu.VMEM((2, _CH * _RPT + _L, ), jnp.float32),
            pltpu.VMEM((2, _CH, _D), jnp.float32),
            pltpu.VMEM((2, _CH, _D), jnp.float32),
            pltpu.VMEM((_L,), jnp.float32),
            pltpu.SemaphoreType.DMA,
            pltpu.SemaphoreType.DMA,
        ],
    )
    return f(entidx, relidx, eE, s2p, dotep, rE, rP)


def kernel(triplets, neg, entityEmb, entityEmbP, relationEmb, relationEmbP):
    h = triplets[:, 0:1].astype(jnp.int32)
    t = triplets[:, 2:3].astype(jnp.int32)
    r = triplets[:, 1].astype(jnp.int32)
    nh = neg[:, :, 0].astype(jnp.int32)
    nt = neg[:, :, 2].astype(jnp.int32)
    ent = jnp.concatenate([h, t, nh, nt], axis=1).reshape(-1)
    s2p, dotep = _p_stats(entityEmb, entityEmbP)
    out = _sc_call(ent, r, entityEmb, s2p, dotep, relationEmb, relationEmbP)
    return jnp.sum(out[:, 0]) / _B


# speedup vs baseline: 2.3385x; 1.0021x over previous
import functools

import jax
import jax.numpy as jnp
from jax import lax
from jax.experimental import pallas as pl
from jax.experimental.pallas import tpu as pltpu
from jax.experimental.pallas import tpu_sc as plsc

_B = 4096
_D = 64
_NC, _NS = 2, 16
_NW = _NC * _NS
_TPW = _B // _NW
_CH = 32
_NCHUNK = _TPW // _CH
_RPT = 8
_MARGIN = 1.0
_L = 16


_BLS = 16384


def _stats_body(e_ref, p_ref, s2p_ref, dot_ref):
    e = e_ref[...]
    p = p_ref[...]
    s2p_ref[...] = jnp.sum(p * p, axis=0)
    dot_ref[...] = jnp.sum(e * p, axis=0)


def _p_stats(e, p):
    eT = jnp.swapaxes(e, 0, 1)
    pT = jnp.swapaxes(p, 0, 1)
    n = eT.shape[1]
    return pl.pallas_call(
        _stats_body,
        out_shape=[jax.ShapeDtypeStruct((n,), jnp.float32),
                   jax.ShapeDtypeStruct((n,), jnp.float32)],
        grid=(pl.cdiv(n, _BLS),),
        in_specs=[pl.BlockSpec((_D, _BLS), lambda j: (0, j)),
                  pl.BlockSpec((_D, _BLS), lambda j: (0, j))],
        out_specs=[pl.BlockSpec((_BLS,), lambda j: (j,)),
                   pl.BlockSpec((_BLS,), lambda j: (j,))],
    )(eT, pT)


def _rsqrt(x):
    i = lax.bitcast_convert_type(x, jnp.int32)
    i = jnp.int32(0x5F3759DF) - lax.shift_right_logical(i, 1)
    y = lax.bitcast_convert_type(i, jnp.float32)
    xh = x * 0.5
    for _ in range(3):
        y = y * (1.5 - xh * y * y)
    return y


def _sumall(v):
    return jnp.full((_L,), jnp.sum(v), jnp.float32)


def _scale(s2):
    norm = s2 * _rsqrt(s2)
    return jnp.minimum(1.0, 1.0 / (norm + 1e-7))


def _sc_body(entidx_hbm, relidx_hbm, eE_hbm, s2p_hbm, dot_hbm, rE_hbm, rP_hbm,
             out_hbm, ei00, ei01, ei10, ei11, ri0, ri1,
             eE_v, s2p_v, dot_v, rE_v, rP_v, out_v, sem0, sem1):
    wid = lax.axis_index("s") * _NC + lax.axis_index("c")
    sems = (sem0, sem1)
    eidx = ((ei00, ei01), (ei10, ei11))
    ridx = (ri0, ri1)
    handles = {}

    def issue(chunk, buf):
        base_t = wid * _TPW + chunk * _CH
        hs = []
        for j in range(2):
            pltpu.sync_copy(
                entidx_hbm.at[pl.ds(base_t * _RPT + j * 128, 128)],
                eidx[buf][j])
            dst_e = eE_v.at[buf, pl.ds(j * 128, 128), :]
            dst_s = s2p_v.at[buf, pl.ds(j * 128, 128)]
            dst_d = dot_v.at[buf, pl.ds(j * 128, 128)]
            hs.append(pltpu.async_copy(eE_hbm.at[eidx[buf][j]], dst_e,
                                       sems[buf]))
            hs.append(pltpu.async_copy(s2p_hbm.at[eidx[buf][j]], dst_s,
                                       sems[buf]))
            hs.append(pltpu.async_copy(dot_hbm.at[eidx[buf][j]], dst_d,
                                       sems[buf]))
        pltpu.sync_copy(relidx_hbm.at[pl.ds(base_t, _CH)], ridx[buf])
        hs.append(pltpu.async_copy(rE_hbm.at[ridx[buf]], rE_v.at[buf],
                                   sems[buf]))
        hs.append(pltpu.async_copy(rP_hbm.at[ridx[buf]], rP_v.at[buf],
                                   sems[buf]))
        handles[buf] = hs

    def row(ref, r0):
        return [ref[r0, pl.ds(c * _L, _L)] for c in range(_D // _L)]

    acc = jnp.zeros((_L,), jnp.float32)
    issue(0, 0)
    for chunk in range(_NCHUNK):
        buf = chunk % 2
        if chunk + 1 < _NCHUNK:
            issue(chunk + 1, 1 - buf)
        for h in handles[buf]:
            h.wait()
        eEb = eE_v.at[buf]
        s2pb, dotb = s2p_v.at[buf], dot_v.at[buf]
        rEb, rPb = rE_v.at[buf], rP_v.at[buf]

        def trip(i, acc):
            s2p8 = s2pb[pl.ds(i * _RPT, _L)]
            dot8 = dotb[pl.ds(i * _RPT, _L)]
            ap8 = _scale(s2p8)
            aE, s_ent = [], []
            for k in range(_RPT):
                r0 = i * _RPT + k
                e = row(eEb, r0)
                s2e = e[0] * e[0]
                for c in range(1, _D // _L):
                    s2e += e[c] * e[c]
                ae = _scale(_sumall(s2e))
                aE.append(ae)
                apk = jnp.full((_L,), ap8[k], jnp.float32)
                dk = jnp.full((_L,), dot8[k], jnp.float32)
                s_ent.append(ae * apk * dk)
            re = row(rEb, i)
            rp = row(rPb, i)
            s2re = re[0] * re[0]
            s2rp = rp[0] * rp[0]
            for c in range(1, _D // _L):
                s2re += re[c] * re[c]
                s2rp += rp[c] * rp[c]
            ar = _scale(_sumall(s2re))
            arp = _scale(_sumall(s2rp))
            rv = [re[c] * ar + 1e-6 for c in range(_D // _L)]
            rps = [rp[c] * arp for c in range(_D // _L)]
            dists = []
            for (hk, tk) in ((0, 1), (2, 5), (3, 6), (4, 7)):
                dS = s_ent[hk] - s_ent[tk]
                eh = row(eEb, i * _RPT + hk)
                et = row(eEb, i * _RPT + tk)
                ds2 = None
                for c in range(_D // _L):
                    dv = (eh[c] * aE[hk] - et[c] * aE[tk]
                          + rps[c] * dS + rv[c])
                    ds2 = dv * dv if ds2 is None else ds2 + dv * dv
                s2 = _sumall(ds2)
                dists.append(s2 * _rsqrt(s2))
            neg_mean = (dists[1] + dists[2] + dists[3]) * (1.0 / 3.0)
            loss = jnp.maximum(dists[0] - neg_mean + _MARGIN, 0.0)
            return acc + loss

        acc = lax.fori_loop(0, _CH, trip, acc)

    out_v[...] = acc
    pltpu.sync_copy(out_v, out_hbm.at[wid])


@jax.jit
def _sc_call(entidx, relidx, eE, s2p, dotep, rE, rP):
    mesh = plsc.VectorSubcoreMesh(core_axis_name="c", subcore_axis_name="s",
                                  num_cores=_NC, num_subcores=_NS)
    f = pl.kernel(
        _sc_body,
        out_type=jax.ShapeDtypeStruct((_NW, _L), jnp.float32),
        mesh=mesh,
        compiler_params=pltpu.CompilerParams(needs_layout_passes=False,
                                             use_tc_tiling_on_sc=False),
        scratch_types=[
            pltpu.VMEM((128,), jnp.int32),
            pltpu.VMEM((128,), jnp.int32),
            pltpu.VMEM((128,), jnp.int32),
            pltpu.VMEM((128,), jnp.int32),
            pltpu.VMEM((_CH,), jnp.int32),
            pltpu.VMEM((_CH,), jnp.int32),
            pltpu.VMEM((2, _CH * _RPT, _D), jnp.float32),
            pltpu.VMEM((2, _CH * _RPT + _L, ), jnp.float32),
            pltpu.VMEM((2, _CH * _RPT + _L, ), jnp.float32),
            pltpu.VMEM((2, _CH, _D), jnp.float32),
            pltpu.VMEM((2, _CH, _D), jnp.float32),
            pltpu.VMEM((_L,), jnp.float32),
            pltpu.SemaphoreType.DMA,
            pltpu.SemaphoreType.DMA,
        ],
    )
    return f(entidx, relidx, eE, s2p, dotep, rE, rP)


def kernel(triplets, neg, entityEmb, entityEmbP, relationEmb, relationEmbP):
    h = triplets[:, 0:1].astype(jnp.int32)
    t = triplets[:, 2:3].astype(jnp.int32)
    r = triplets[:, 1].astype(jnp.int32)
    nh = neg[:, :, 0].astype(jnp.int32)
    nt = neg[:, :, 2].astype(jnp.int32)
    ent = jnp.concatenate([h, t, nh, nt], axis=1).reshape(-1)
    s2p, dotep = _p_stats(entityEmb, entityEmbP)
    out = _sc_call(ent, r, entityEmb, s2p, dotep, relationEmb, relationEmbP)
    return jnp.sum(out[:, 0]) / _B
